# Initial kernel scaffold; baseline (speedup 1.0000x reference)
import jax
import jax.numpy as jnp
from jax.experimental import pallas as pl


def kernel(x, src0, dst0, src1, dst1, src2, dst2, embed_W, Wself0, Wneigh0, b0, Wself1, Wneigh1, b1, Wself2, Wneigh2, b2, fcW, fcb):
    raise NotImplementedError("write your pallas kernel here")


# speedup vs baseline: 2.0574x; 2.0574x over previous
import functools

import jax
import jax.numpy as jnp
from jax import lax
from jax.experimental import pallas as pl
from jax.experimental.pallas import tpu as pltpu
from jax.experimental.pallas import tpu_sc as plsc

F32 = jnp.float32

_N0, _N1, _N2, _N3 = 100000, 25000, 6400, 1024
_E0, _E1, _E2 = 400000, 102400, 16384
_F_IN, _H, _C = 512, 128, 128

_NC, _NS = 2, 16
_B = 128


def _mm_body(x_ref, w_ref, o_ref):
    o_ref[...] = jnp.dot(x_ref[...], w_ref[...], preferred_element_type=F32)


def _embed(x, w_t):
    blk = 2000
    grid = _N0 // blk
    return pl.pallas_call(
        _mm_body,
        grid=(grid,),
        in_specs=[
            pl.BlockSpec((blk, _F_IN), lambda i: (i, 0)),
            pl.BlockSpec((_F_IN, _H), lambda i: (0, 0)),
        ],
        out_specs=pl.BlockSpec((blk, _H), lambda i: (i, 0)),
        out_shape=jax.ShapeDtypeStruct((_N0, _H), F32),
    )(x, w_t)


def _layer_body(hd_ref, sm_ref, dg_ref, ws_ref, wn_ref, b_ref, o_ref):
    deg = jnp.maximum(dg_ref[...][:, :1], 1.0)
    neigh = sm_ref[...] / deg
    acc = (jnp.dot(hd_ref[...], ws_ref[...], preferred_element_type=F32)
           + jnp.dot(neigh, wn_ref[...], preferred_element_type=F32)
           + b_ref[...])
    o_ref[...] = jnp.maximum(acc, 0.0)


def _layer_fc_body(hd_ref, sm_ref, dg_ref, ws_ref, wn_ref, b_ref,
                   fw_ref, fb_ref, o_ref):
    deg = jnp.maximum(dg_ref[...][:, :1], 1.0)
    neigh = sm_ref[...] / deg
    acc = (jnp.dot(hd_ref[...], ws_ref[...], preferred_element_type=F32)
           + jnp.dot(neigh, wn_ref[...], preferred_element_type=F32)
           + b_ref[...])
    o_ref[...] = jnp.dot(acc, fw_ref[...], preferred_element_type=F32) + fb_ref[...]


def _layer(h_prev, sums, deg, ws_t, wn_t, b, n_out, blk):
    grid = n_out // blk
    return pl.pallas_call(
        _layer_body,
        grid=(grid,),
        in_specs=[
            pl.BlockSpec((blk, _H), lambda i: (i, 0)),
            pl.BlockSpec((blk, _H), lambda i: (i, 0)),
            pl.BlockSpec((blk, 16), lambda i: (i, 0)),
            pl.BlockSpec((_H, _H), lambda i: (0, 0)),
            pl.BlockSpec((_H, _H), lambda i: (0, 0)),
            pl.BlockSpec((1, _H), lambda i: (0, 0)),
        ],
        out_specs=pl.BlockSpec((blk, _H), lambda i: (i, 0)),
        out_shape=jax.ShapeDtypeStruct((n_out, _H), F32),
    )(h_prev, sums, deg, ws_t, wn_t, b)


def _layer_fc(h_prev, sums, deg, ws_t, wn_t, b, fw_t, fb, n_out):
    return pl.pallas_call(
        _layer_fc_body,
        grid=(1,),
        in_specs=[
            pl.BlockSpec((n_out, _H), lambda i: (0, 0)),
            pl.BlockSpec((n_out, _H), lambda i: (0, 0)),
            pl.BlockSpec((n_out, 16), lambda i: (0, 0)),
            pl.BlockSpec((_H, _H), lambda i: (0, 0)),
            pl.BlockSpec((_H, _H), lambda i: (0, 0)),
            pl.BlockSpec((1, _H), lambda i: (0, 0)),
            pl.BlockSpec((_H, _C), lambda i: (0, 0)),
            pl.BlockSpec((1, _C), lambda i: (0, 0)),
        ],
        out_specs=pl.BlockSpec((n_out, _C), lambda i: (0, 0)),
        out_shape=jax.ShapeDtypeStruct((n_out, _C), F32),
    )(h_prev, sums, deg, ws_t, wn_t, b, fw_t, fb)


_MESH = plsc.VectorSubcoreMesh(core_axis_name="c", subcore_axis_name="s",
                               num_cores=_NC, num_subcores=_NS)


def _make_sum_agg(e_pad, split, rng, alloc, zspan, trash, wout, n_out):
    chunks = e_pad // (_NS * _B)

    @functools.partial(
        pl.kernel,
        out_type=jax.ShapeDtypeStruct((n_out, _H), F32),
        mesh=_MESH,
        scratch_types=[
            pltpu.VMEM((_B,), jnp.int32),
            pltpu.VMEM((_B,), jnp.int32),
            pltpu.VMEM((_B,), jnp.int32),
            pltpu.VMEM((_B, _H), F32),
            pltpu.VMEM_SHARED((alloc, _H), F32),
            pltpu.SemaphoreType.DMA,
        ],
    )
    def agg(h_hbm, src_hbm, dst_hbm, sums_out,
            idx_src, idx_dst, idx_loc, rows, sums_sh, sem):
        c = lax.axis_index("c")
        s = lax.axis_index("s")

        def zrow(i, _):
            def zcol(j, _):
                rows[i, pl.ds(j * 16, 16)] = jnp.zeros((16,), F32)
                return 0
            lax.fori_loop(0, _H // 16, zcol, 0)
            return 0
        lax.fori_loop(0, 8, zrow, 0)

        def zshared(t, _):
            off = s * zspan + t * 8
            pltpu.sync_copy(rows.at[pl.ds(0, 8)], sums_sh.at[pl.ds(off, 8)])
            return 0
        lax.fori_loop(0, zspan // 8, zshared, 0)

        plsc.subcore_barrier()

        lo = c * split

        def step(j, _):
            base = (s * chunks + j) * _B
            pltpu.sync_copy(src_hbm.at[pl.ds(base, _B)], idx_src)
            pltpu.sync_copy(dst_hbm.at[pl.ds(base, _B)], idx_dst)
            pltpu.async_copy(h_hbm.at[idx_src], rows, sem).wait()
            def loc16(k, _):
                d = idx_dst[pl.ds(k * 16, 16)]
                l = d - lo
                ok = (l >= 0) & (l < rng)
                idx_loc[pl.ds(k * 16, 16)] = jnp.where(ok, l, trash)
                return 0
            lax.fori_loop(0, _B // 16, loc16, 0)
            pltpu.sync_copy(rows, sums_sh.at[idx_loc], add=True)
            return 0
        lax.fori_loop(0, chunks, step, 0)

        plsc.subcore_barrier()

        off = c * split + s * wout
        pltpu.sync_copy(sums_sh.at[pl.ds(s * wout, wout)],
                        sums_out.at[pl.ds(off, wout)])

    return agg


def _make_deg_agg(e_pad, split, rng, alloc, zspan, trash, wout, n_out):
    chunks = e_pad // (_NS * _B)

    @functools.partial(
        pl.kernel,
        out_type=jax.ShapeDtypeStruct((n_out, 16), F32),
        mesh=_MESH,
        scratch_types=[
            pltpu.VMEM((_B,), jnp.int32),
            pltpu.VMEM((_B,), jnp.int32),
            pltpu.VMEM((_B, 16), F32),
            pltpu.VMEM_SHARED((alloc, 16), F32),
        ],
    )
    def agg(dst_hbm, deg_out, idx_dst, idx_loc, ones_b, deg_sh):
        c = lax.axis_index("c")
        s = lax.axis_index("s")

        def zrow(i, _):
            ones_b[i, :] = jnp.zeros((16,), F32)
            return 0
        lax.fori_loop(0, 8, zrow, 0)

        def zshared(t, _):
            off = s * zspan + t * 8
            pltpu.sync_copy(ones_b.at[pl.ds(0, 8)], deg_sh.at[pl.ds(off, 8)])
            return 0
        lax.fori_loop(0, zspan // 8, zshared, 0)

        def fill_ones(i, _):
            ones_b[i, :] = jnp.ones((16,), F32)
            return 0
        lax.fori_loop(0, _B, fill_ones, 0)

        plsc.subcore_barrier()

        lo = c * split

        def step(j, _):
            base = (s * chunks + j) * _B
            pltpu.sync_copy(dst_hbm.at[pl.ds(base, _B)], idx_dst)
            def loc16(k, _):
                d = idx_dst[pl.ds(k * 16, 16)]
                l = d - lo
                ok = (l >= 0) & (l < rng)
                idx_loc[pl.ds(k * 16, 16)] = jnp.where(ok, l, trash)
                return 0
            lax.fori_loop(0, _B // 16, loc16, 0)
            pltpu.sync_copy(ones_b, deg_sh.at[idx_loc], add=True)
            return 0
        lax.fori_loop(0, chunks, step, 0)

        plsc.subcore_barrier()

        off = c * split + s * wout
        pltpu.sync_copy(deg_sh.at[pl.ds(s * wout, wout)],
                        deg_out.at[pl.ds(off, wout)])

    return agg


_CFG0 = (409600, 12544, 12544, 12672, 792, 12600, 784, 25088)
_CFG1 = (_E1, 3200, 3200, 3328, 208, 3264, 200, _N2)
_CFG2 = (_E2, 512, 512, 640, 40, 576, 32, _N3)
_SUM0, _DEG0 = _make_sum_agg(*_CFG0), _make_deg_agg(*_CFG0)
_SUM1, _DEG1 = _make_sum_agg(*_CFG1), _make_deg_agg(*_CFG1)
_SUM2, _DEG2 = _make_sum_agg(*_CFG2), _make_deg_agg(*_CFG2)


@jax.jit
def kernel(x, src0, dst0, src1, dst1, src2, dst2, embed_W,
           Wself0, Wneigh0, b0, Wself1, Wneigh1, b1, Wself2, Wneigh2, b2,
           fcW, fcb):
    pad = 409600 - _E0
    src0p = jnp.concatenate([src0, jnp.zeros((pad,), jnp.int32)])
    dst0p = jnp.concatenate([dst0, jnp.full((pad,), _N1, jnp.int32)])

    h0 = _embed(x, embed_W.T)

    s0 = _SUM0(h0, src0p, dst0p)
    d0 = _DEG0(dst0p)
    h1 = _layer(h0, s0, d0, Wself0.T, Wneigh0.T, b0.reshape(1, -1),
                n_out=25088, blk=784)

    s1 = _SUM1(h1, src1, dst1)
    d1 = _DEG1(dst1)
    h2 = _layer(h1, s1, d1, Wself1.T, Wneigh1.T, b1.reshape(1, -1),
                n_out=_N2, blk=800)

    s2 = _SUM2(h2, src2, dst2)
    d2 = _DEG2(dst2)
    return _layer_fc(h2, s2, d2, Wself2.T, Wneigh2.T, b2.reshape(1, -1),
                     fcW.T, fcb.reshape(1, -1), n_out=_N3)


# speedup vs baseline: 2.7974x; 1.3597x over previous
import functools

import jax
import jax.numpy as jnp
from jax import lax
from jax.experimental import pallas as pl
from jax.experimental.pallas import tpu as pltpu
from jax.experimental.pallas import tpu_sc as plsc

F32 = jnp.float32

_N0, _N1, _N2, _N3 = 100000, 25000, 6400, 1024
_E0, _E1, _E2 = 400000, 102400, 16384
_F_IN, _H, _C = 512, 128, 128

_NC, _NS = 2, 16
_B = 128


def _mm_body(x_ref, w_ref, o_ref):
    o_ref[...] = jnp.dot(x_ref[...], w_ref[...], preferred_element_type=F32)


def _embed(x, w_t):
    blk = 2000
    grid = _N0 // blk
    return pl.pallas_call(
        _mm_body,
        grid=(grid,),
        in_specs=[
            pl.BlockSpec((blk, _F_IN), lambda i: (i, 0)),
            pl.BlockSpec((_F_IN, _H), lambda i: (0, 0)),
        ],
        out_specs=pl.BlockSpec((blk, _H), lambda i: (i, 0)),
        out_shape=jax.ShapeDtypeStruct((_N0, _H), F32),
    )(x, w_t)


def _layer_body(hd_ref, sm_ref, dg_ref, ws_ref, wn_ref, b_ref, o_ref):
    deg = jnp.maximum(dg_ref[...][:, :1], 1.0)
    neigh = sm_ref[...] / deg
    acc = (jnp.dot(hd_ref[...], ws_ref[...], preferred_element_type=F32)
           + jnp.dot(neigh, wn_ref[...], preferred_element_type=F32)
           + b_ref[...])
    o_ref[...] = jnp.maximum(acc, 0.0)


def _layer_fc_body(hd_ref, sm_ref, dg_ref, ws_ref, wn_ref, b_ref,
                   fw_ref, fb_ref, o_ref):
    deg = jnp.maximum(dg_ref[...][:, :1], 1.0)
    neigh = sm_ref[...] / deg
    acc = (jnp.dot(hd_ref[...], ws_ref[...], preferred_element_type=F32)
           + jnp.dot(neigh, wn_ref[...], preferred_element_type=F32)
           + b_ref[...])
    o_ref[...] = jnp.dot(acc, fw_ref[...], preferred_element_type=F32) + fb_ref[...]


def _layer(h_prev, sums, deg, ws_t, wn_t, b, n_out, blk):
    grid = n_out // blk
    return pl.pallas_call(
        _layer_body,
        grid=(grid,),
        in_specs=[
            pl.BlockSpec((blk, _H), lambda i: (i, 0)),
            pl.BlockSpec((blk, _H), lambda i: (i, 0)),
            pl.BlockSpec((blk, 16), lambda i: (i, 0)),
            pl.BlockSpec((_H, _H), lambda i: (0, 0)),
            pl.BlockSpec((_H, _H), lambda i: (0, 0)),
            pl.BlockSpec((1, _H), lambda i: (0, 0)),
        ],
        out_specs=pl.BlockSpec((blk, _H), lambda i: (i, 0)),
        out_shape=jax.ShapeDtypeStruct((n_out, _H), F32),
    )(h_prev, sums, deg, ws_t, wn_t, b)


def _layer_fc(h_prev, sums, deg, ws_t, wn_t, b, fw_t, fb, n_out):
    return pl.pallas_call(
        _layer_fc_body,
        grid=(1,),
        in_specs=[
            pl.BlockSpec((n_out, _H), lambda i: (0, 0)),
            pl.BlockSpec((n_out, _H), lambda i: (0, 0)),
            pl.BlockSpec((n_out, 16), lambda i: (0, 0)),
            pl.BlockSpec((_H, _H), lambda i: (0, 0)),
            pl.BlockSpec((_H, _H), lambda i: (0, 0)),
            pl.BlockSpec((1, _H), lambda i: (0, 0)),
            pl.BlockSpec((_H, _C), lambda i: (0, 0)),
            pl.BlockSpec((1, _C), lambda i: (0, 0)),
        ],
        out_specs=pl.BlockSpec((n_out, _C), lambda i: (0, 0)),
        out_shape=jax.ShapeDtypeStruct((n_out, _C), F32),
    )(h_prev, sums, deg, ws_t, wn_t, b, fw_t, fb)


_MESH = plsc.VectorSubcoreMesh(core_axis_name="c", subcore_axis_name="s",
                               num_cores=_NC, num_subcores=_NS)


def _make_sum_agg(chunks, split, rng, alloc, zspan, trash, wout, n_out,
                  B=96, nbuf=2):
    assert chunks % nbuf == 0 and chunks >= 2 * nbuf

    @functools.partial(
        pl.kernel,
        out_type=jax.ShapeDtypeStruct((n_out, _H), F32),
        mesh=_MESH,
        scratch_types=[
            pltpu.VMEM((nbuf, B), jnp.int32),
            pltpu.VMEM((nbuf, B), jnp.int32),
            pltpu.VMEM((nbuf, B), jnp.int32),
            pltpu.VMEM((nbuf, B, _H), F32),
            pltpu.VMEM_SHARED((alloc, _H), F32),
        ] + [pltpu.SemaphoreType.DMA] * (3 * nbuf),
    )
    def agg(h_hbm, src_hbm, dst_hbm, sums_out,
            idx_src, idx_dst, idx_loc, rows, sums_sh, *sems):
        gsem, ssem, isem = sems[:nbuf], sems[nbuf:2 * nbuf], sems[2 * nbuf:]
        c = lax.axis_index("c")
        s = lax.axis_index("s")

        def zrow(i, _):
            def zcol(j, _):
                rows[0, i, pl.ds(j * 16, 16)] = jnp.zeros((16,), F32)
                return 0
            lax.fori_loop(0, _H // 16, zcol, 0)
            return 0
        lax.fori_loop(0, 8, zrow, 0)

        def zshared(t, _):
            off = s * zspan + t * 8
            pltpu.sync_copy(rows.at[0].at[pl.ds(0, 8)],
                            sums_sh.at[pl.ds(off, 8)])
            return 0
        lax.fori_loop(0, zspan // 8, zshared, 0)

        plsc.subcore_barrier()

        lo = c * split
        base0 = s * chunks * B

        def compute_loc(b):
            def loc16(k, _):
                d = idx_dst[b, pl.ds(k * 16, 16)]
                l = d - lo
                ok = (l >= 0) & (l < rng)
                idx_loc[b, pl.ds(k * 16, 16)] = jnp.where(ok, l, trash)
                return 0
            lax.fori_loop(0, B // 16, loc16, 0)

        def fire_gather(b):
            pltpu.async_copy(h_hbm.at[idx_src.at[b]], rows.at[b], gsem[b])

        def wait_gather(b):
            pltpu.make_async_copy(h_hbm.at[idx_src.at[b]], rows.at[b],
                                  gsem[b]).wait()

        def fire_scatter(b):
            pltpu.async_copy(rows.at[b], sums_sh.at[idx_loc.at[b]], ssem[b],
                             add=True)

        def wait_scatter(b):
            pltpu.make_async_copy(rows.at[b], sums_sh.at[idx_loc.at[b]],
                                  ssem[b]).wait()

        for b in range(nbuf):
            base = base0 + b * B
            pltpu.sync_copy(src_hbm.at[pl.ds(base, B)], idx_src.at[b])
            pltpu.sync_copy(dst_hbm.at[pl.ds(base, B)], idx_dst.at[b])
            fire_gather(b)

        def body(g, _):
            for b in range(nbuf):
                nbase = base0 + (nbuf * g + b + nbuf) * B
                wait_gather(b)
                compute_loc(b)
                fire_scatter(b)
                pltpu.async_copy(src_hbm.at[pl.ds(nbase, B)],
                                 idx_src.at[b], isem[b])
                pltpu.async_copy(dst_hbm.at[pl.ds(nbase, B)],
                                 idx_dst.at[b], isem[b])
                wait_scatter(b)
                pltpu.make_async_copy(src_hbm.at[pl.ds(nbase, B)],
                                      idx_src.at[b], isem[b]).wait()
                pltpu.make_async_copy(dst_hbm.at[pl.ds(nbase, B)],
                                      idx_dst.at[b], isem[b]).wait()
                fire_gather(b)
            return 0
        lax.fori_loop(0, chunks // nbuf - 1, body, 0)

        for b in range(nbuf):
            wait_gather(b)
            compute_loc(b)
            fire_scatter(b)
            wait_scatter(b)

        plsc.subcore_barrier()

        off = c * split + s * wout
        pltpu.sync_copy(sums_sh.at[pl.ds(s * wout, wout)],
                        sums_out.at[pl.ds(off, wout)])

    return agg


def _make_deg_agg(chunks, split, rng, alloc, zspan, trash, wout, n_out):
    assert chunks % 4 == 0 and chunks >= 8
    nbuf = 4

    @functools.partial(
        pl.kernel,
        out_type=jax.ShapeDtypeStruct((n_out, 16), F32),
        mesh=_MESH,
        scratch_types=[
            pltpu.VMEM((nbuf, _B), jnp.int32),
            pltpu.VMEM((nbuf, _B), jnp.int32),
            pltpu.VMEM((_B, 16), F32),
            pltpu.VMEM_SHARED((alloc, 16), F32),
        ] + [pltpu.SemaphoreType.DMA] * (2 * nbuf),
    )
    def agg(dst_hbm, deg_out, idx_dst, idx_loc, ones_b, deg_sh, *sems):
        ssem, isem = sems[:nbuf], sems[nbuf:]
        c = lax.axis_index("c")
        s = lax.axis_index("s")

        def zrow(i, _):
            ones_b[i, :] = jnp.zeros((16,), F32)
            return 0
        lax.fori_loop(0, 8, zrow, 0)

        def zshared(t, _):
            off = s * zspan + t * 8
            pltpu.sync_copy(ones_b.at[pl.ds(0, 8)], deg_sh.at[pl.ds(off, 8)])
            return 0
        lax.fori_loop(0, zspan // 8, zshared, 0)

        def fill_ones(i, _):
            ones_b[i, :] = jnp.ones((16,), F32)
            return 0
        lax.fori_loop(0, _B, fill_ones, 0)

        plsc.subcore_barrier()

        lo = c * split
        base0 = s * chunks * _B

        def compute_loc(b):
            def loc16(k, _):
                d = idx_dst[b, pl.ds(k * 16, 16)]
                l = d - lo
                ok = (l >= 0) & (l < rng)
                idx_loc[b, pl.ds(k * 16, 16)] = jnp.where(ok, l, trash)
                return 0
            lax.fori_loop(0, _B // 16, loc16, 0)

        def fire_scatter(b):
            pltpu.async_copy(ones_b, deg_sh.at[idx_loc.at[b]], ssem[b],
                             add=True)

        def wait_scatter(b):
            pltpu.make_async_copy(ones_b, deg_sh.at[idx_loc.at[b]],
                                  ssem[b]).wait()

        for b in range(nbuf):
            base = base0 + b * _B
            pltpu.sync_copy(dst_hbm.at[pl.ds(base, _B)], idx_dst.at[b])
            compute_loc(b)
            fire_scatter(b)

        def body(g, _):
            for b in range(nbuf):
                nbase = base0 + (4 * g + b + 4) * _B
                pltpu.async_copy(dst_hbm.at[pl.ds(nbase, _B)],
                                 idx_dst.at[b], isem[b])
                wait_scatter(b)
                pltpu.make_async_copy(dst_hbm.at[pl.ds(nbase, _B)],
                                      idx_dst.at[b], isem[b]).wait()
                compute_loc(b)
                fire_scatter(b)
            return 0
        lax.fori_loop(0, chunks // 4 - 1, body, 0)

        for b in range(nbuf):
            wait_scatter(b)

        plsc.subcore_barrier()

        off = c * split + s * wout
        pltpu.sync_copy(deg_sh.at[pl.ds(s * wout, wout)],
                        deg_out.at[pl.ds(off, wout)])

    return agg


_CFG0 = (12544, 12544, 12672, 792, 12600, 784, 25088)
_CFG1 = (3200, 3200, 3328, 208, 3264, 200, _N2)
_CFG2 = (512, 512, 640, 40, 576, 32, _N3)
_SUM0, _DEG0 = _make_sum_agg(262, *_CFG0), _make_deg_agg(196, *_CFG0)
_SUM1, _DEG1 = _make_sum_agg(68, *_CFG1), _make_deg_agg(52, *_CFG1)
_SUM2, _DEG2 = _make_sum_agg(12, *_CFG2), _make_deg_agg(8, *_CFG2)


@jax.jit
def kernel(x, src0, dst0, src1, dst1, src2, dst2, embed_W,
           Wself0, Wneigh0, b0, Wself1, Wneigh1, b1, Wself2, Wneigh2, b2,
           fcW, fcb):
    def _pad_edges(src, dst, total, dump):
        pad = total - src.shape[0]
        return (jnp.concatenate([src, jnp.zeros((pad,), jnp.int32)]),
                jnp.concatenate([dst, jnp.full((pad,), dump, jnp.int32)]))

    src0p, dst0p = _pad_edges(src0, dst0, 402432, _N1)
    src1p, dst1p = _pad_edges(src1, dst1, 106496, _N2)
    src2p, dst2p = _pad_edges(src2, dst2, 18432, _N3)

    d0 = _DEG0(dst0p)
    d1 = _DEG1(dst1p)
    d2 = _DEG2(dst2p)

    h0 = _embed(x, embed_W.T)

    s0 = _SUM0(h0, src0p, dst0p)
    h1 = _layer(h0, s0, d0, Wself0.T, Wneigh0.T, b0.reshape(1, -1),
                n_out=25088, blk=784)

    s1 = _SUM1(h1, src1p, dst1p)
    h2 = _layer(h1, s1, d1, Wself1.T, Wneigh1.T, b1.reshape(1, -1),
                n_out=_N2, blk=800)

    s2 = _SUM2(h2, src2p, dst2p)
    return _layer_fc(h2, s2, d2, Wself2.T, Wneigh2.T, b2.reshape(1, -1),
                     fcW.T, fcb.reshape(1, -1), n_out=_N3)
